# baseline (speedup 1.0000x reference)
import jax
import jax.numpy as jnp
from jax import lax
from jax.experimental import pallas as pl
from jax.experimental.pallas import tpu as pltpu
from jax.experimental.pallas import tpu_sc as plsc

_NC = 2
_NS = 16
_NW = _NC * _NS
_L = 16


def _sc_body(table_hbm, idx_hbm, out_hbm, table_v, idx_v, out_v):
    n = idx_v.shape[0]
    wid = lax.axis_index("s") * _NC + lax.axis_index("c")
    base = wid * n
    pltpu.sync_copy(table_hbm, table_v)
    pltpu.sync_copy(idx_hbm.at[pl.ds(base, n)], idx_v)

    tv = table_v[...]
    dnums = lax.GatherDimensionNumbers(
        offset_dims=(), collapsed_slice_dims=(0,), start_index_map=(0,))

    unroll = 16
    chunk = unroll * _L

    def step(i, carry):
        off = i * chunk
        for j in range(unroll):
            o = off + j * _L
            x = idx_v[pl.ds(o, _L)]
            out_v[pl.ds(o, _L)] = lax.gather(
                tv, x[:, None], dnums, (1,),
                mode=lax.GatherScatterMode.PROMISE_IN_BOUNDS)
        return carry

    lax.fori_loop(0, n // chunk, step, 0)
    pltpu.sync_copy(out_v, out_hbm.at[pl.ds(base, n)])


def kernel(inputs, table):
    B, S = inputs.shape
    n_total = B * S
    per_w = n_total // _NW
    tr, r, tc, c = S // 8, 8, B // 128, 128
    flat = (inputs.T.astype(jnp.int32)
            .reshape(tr, r, tc, c)
            .transpose(0, 2, 1, 3)
            .reshape(n_total))
    t01 = jnp.pad(table.reshape(-1).astype(jnp.float32), (0, _L - 2))
    mesh = plsc.VectorSubcoreMesh(core_axis_name="c", subcore_axis_name="s",
                                  num_cores=_NC, num_subcores=_NS)
    f = pl.kernel(
        _sc_body,
        out_type=jax.ShapeDtypeStruct((n_total,), jnp.float32),
        mesh=mesh,
        scratch_types=[
            pltpu.VMEM((_L,), jnp.float32),
            pltpu.VMEM((per_w,), jnp.int32),
            pltpu.VMEM((per_w,), jnp.float32),
        ],
    )
    out_flat = f(t01, flat)
    return (out_flat.reshape(tr, tc, r, c)
            .transpose(0, 2, 1, 3)
            .reshape(S, B).T)


# speedup vs baseline: 1.0388x; 1.0388x over previous
import jax
import jax.numpy as jnp
from jax import lax
from jax.experimental import pallas as pl
from jax.experimental.pallas import tpu as pltpu
from jax.experimental.pallas import tpu_sc as plsc

_NC = 2
_NS = 16
_NW = _NC * _NS
_L = 16


def _sc_body(table_hbm, idx_hbm, out_hbm, table_v, idx_v, out_v):
    n = idx_v.shape[0]
    wid = lax.axis_index("s") * _NC + lax.axis_index("c")
    base = wid * n
    pltpu.sync_copy(table_hbm, table_v.at[pl.ds(0, 2)])
    pltpu.sync_copy(idx_hbm.at[pl.ds(base, n)], idx_v)

    tv = table_v[...]
    dnums = lax.GatherDimensionNumbers(
        offset_dims=(), collapsed_slice_dims=(0,), start_index_map=(0,))

    unroll = 16
    chunk = unroll * _L

    def step(i, carry):
        off = i * chunk
        for j in range(unroll):
            o = off + j * _L
            x = idx_v[pl.ds(o, _L)]
            out_v[pl.ds(o, _L)] = lax.gather(
                tv, x[:, None], dnums, (1,),
                mode=lax.GatherScatterMode.PROMISE_IN_BOUNDS)
        return carry

    lax.fori_loop(0, n // chunk, step, 0)
    pltpu.sync_copy(out_v, out_hbm.at[pl.ds(base, n)])


def kernel(inputs, table):
    B, S = inputs.shape
    n_total = B * S
    per_w = n_total // _NW
    tr, r, tc, c = S // 8, 8, B // 128, 128
    flat = (inputs.T.astype(jnp.int32)
            .reshape(tr, r, tc, c)
            .transpose(0, 2, 1, 3)
            .reshape(n_total))
    t01 = table.reshape(-1).astype(jnp.float32)
    mesh = plsc.VectorSubcoreMesh(core_axis_name="c", subcore_axis_name="s",
                                  num_cores=_NC, num_subcores=_NS)
    f = pl.kernel(
        _sc_body,
        out_type=jax.ShapeDtypeStruct((n_total,), jnp.float32),
        mesh=mesh,
        scratch_types=[
            pltpu.VMEM((_L,), jnp.float32),
            pltpu.VMEM((per_w,), jnp.int32),
            pltpu.VMEM((per_w,), jnp.float32),
        ],
    )
    out_flat = f(t01, flat)
    return (out_flat.reshape(tr, tc, r, c)
            .transpose(0, 2, 1, 3)
            .reshape(S, B).T)


# speedup vs baseline: 1.0830x; 1.0426x over previous
import jax
import jax.numpy as jnp
from jax import lax
from jax.experimental import pallas as pl
from jax.experimental.pallas import tpu as pltpu
from jax.experimental.pallas import tpu_sc as plsc

_NC = 2
_NS = 16
_NW = _NC * _NS
_L = 16


def _sc_body(table_hbm, idx_hbm, out_hbm, table_v,
             idx_v0, idx_v1, out_v0, out_v1, sem_i0, sem_i1, sem_o0, sem_o1):
    h = idx_v0.shape[0]
    wid = lax.axis_index("s") * _NC + lax.axis_index("c")
    base = wid * (2 * h)
    cp0 = pltpu.async_copy(idx_hbm.at[pl.ds(base, h)], idx_v0, sem_i0)
    cp1 = pltpu.async_copy(idx_hbm.at[pl.ds(base + h, h)], idx_v1, sem_i1)
    pltpu.sync_copy(table_hbm, table_v.at[pl.ds(0, 2)])

    tv = table_v[...]
    dnums = lax.GatherDimensionNumbers(
        offset_dims=(), collapsed_slice_dims=(0,), start_index_map=(0,))

    unroll = 16
    chunk = unroll * _L

    def half(idx_v, out_v):
        def step(i, carry):
            off = i * chunk
            for j in range(unroll):
                o = off + j * _L
                x = idx_v[pl.ds(o, _L)]
                out_v[pl.ds(o, _L)] = lax.gather(
                    tv, x[:, None], dnums, (1,),
                    mode=lax.GatherScatterMode.PROMISE_IN_BOUNDS)
            return carry
        lax.fori_loop(0, h // chunk, step, 0)

    cp0.wait()
    half(idx_v0, out_v0)
    wr0 = pltpu.async_copy(out_v0, out_hbm.at[pl.ds(base, h)], sem_o0)
    cp1.wait()
    half(idx_v1, out_v1)
    wr1 = pltpu.async_copy(out_v1, out_hbm.at[pl.ds(base + h, h)], sem_o1)
    wr0.wait()
    wr1.wait()


def kernel(inputs, table):
    B, S = inputs.shape
    n_total = B * S
    per_w = n_total // _NW
    tr, r, tc, c = S // 8, 8, B // 128, 128
    flat = (inputs.T.astype(jnp.int32)
            .reshape(tr, r, tc, c)
            .transpose(0, 2, 1, 3)
            .reshape(n_total))
    t01 = table.reshape(-1).astype(jnp.float32)
    mesh = plsc.VectorSubcoreMesh(core_axis_name="c", subcore_axis_name="s",
                                  num_cores=_NC, num_subcores=_NS)
    f = pl.kernel(
        _sc_body,
        out_type=jax.ShapeDtypeStruct((n_total,), jnp.float32),
        mesh=mesh,
        scratch_types=[
            pltpu.VMEM((_L,), jnp.float32),
            pltpu.VMEM((per_w // 2,), jnp.int32),
            pltpu.VMEM((per_w // 2,), jnp.int32),
            pltpu.VMEM((per_w // 2,), jnp.float32),
            pltpu.VMEM((per_w // 2,), jnp.float32),
            pltpu.SemaphoreType.DMA,
            pltpu.SemaphoreType.DMA,
            pltpu.SemaphoreType.DMA,
            pltpu.SemaphoreType.DMA,
        ],
    )
    out_flat = f(t01, flat)
    return (out_flat.reshape(tr, tc, r, c)
            .transpose(0, 2, 1, 3)
            .reshape(S, B).T)
